# baseline (speedup 1.0000x reference)
import functools

import jax
import jax.numpy as jnp
from jax import lax
from jax.experimental import pallas as pl
from jax.experimental.pallas import tpu as pltpu
from jax.experimental.pallas import tpu_sc as plsc

B = 4096
L = 50
D = 32
H = 128
C = 32

NC = 2
NS = 16
NW = NC * NS
BPW = B // NW
G = 2
CHUNK = G * L
NCH = BPW // G
INV_L = 1.0 / L


def _sc_pool(x_r, W_word, W_bi, W_tri):
    mesh = plsc.VectorSubcoreMesh(core_axis_name="c", subcore_axis_name="s")

    @functools.partial(
        pl.kernel,
        out_type=jax.ShapeDtypeStruct((B, 3 * D), jnp.float32),
        mesh=mesh,
        compiler_params=pltpu.CompilerParams(use_tc_tiling_on_sc=False),
        scratch_types=[
            pltpu.VMEM((NCH, CHUNK), jnp.int32),
            pltpu.VMEM((CHUNK, D), jnp.float32),
            pltpu.VMEM((CHUNK, D), jnp.float32),
            pltpu.VMEM((CHUNK, D), jnp.float32),
            pltpu.VMEM((BPW, 3 * D), jnp.float32),
            pltpu.SemaphoreType.DMA,
            pltpu.SemaphoreType.DMA,
            pltpu.SemaphoreType.DMA,
        ],
    )
    def k(x_hbm, ww_hbm, wb_hbm, wt_hbm, out_hbm,
          idx_v, bufw, bufb, buft, pooled, semw, semb, semt):
        wid = lax.axis_index("s") * NC + lax.axis_index("c")
        pltpu.sync_copy(x_hbm.at[wid], idx_v)

        def chunk_body(c, carry):
            idx = idx_v.at[c]
            cw = pltpu.async_copy(ww_hbm.at[idx], bufw, semw)
            cb = pltpu.async_copy(wb_hbm.at[idx], bufb, semb)
            ct = pltpu.async_copy(wt_hbm.at[idx], buft, semt)
            cw.wait()
            cb.wait()
            ct.wait()
            for t, buf in enumerate((bufw, bufb, buft)):
                for g in range(G):
                    def acc_body(l, a):
                        r = g * L + l
                        return (a[0] + buf[r, pl.ds(0, 16)],
                                a[1] + buf[r, pl.ds(16, 16)])
                    a0, a1 = lax.fori_loop(
                        0, L, acc_body,
                        (jnp.zeros((16,), jnp.float32),
                         jnp.zeros((16,), jnp.float32)))
                    row = c * G + g
                    pooled[row, pl.ds(t * D, 16)] = a0 * INV_L
                    pooled[row, pl.ds(t * D + 16, 16)] = a1 * INV_L
            return carry

        lax.fori_loop(0, NCH, chunk_body, 0)
        pltpu.sync_copy(pooled, out_hbm.at[pl.ds(wid * BPW, BPW)])

    return k(x_r, W_word, W_bi, W_tri)


def _mlp(pooled, W1, b1, W2, b2):
    def body(p_ref, w1_ref, b1_ref, w2_ref, b2_ref, o_ref):
        h = jnp.dot(p_ref[...], w1_ref[...],
                    preferred_element_type=jnp.float32) + b1_ref[...]
        o = jnp.dot(h, w2_ref[...],
                    preferred_element_type=jnp.float32) + b2_ref[...]
        o_ref[...] = jnp.maximum(o, 0.0)

    return pl.pallas_call(
        body,
        out_shape=jax.ShapeDtypeStruct((B, C), jnp.float32),
    )(pooled, W1, b1.reshape(1, H), W2, b2.reshape(1, C))


def kernel(x, W_word, W_bi, W_tri, W1, b1, W2, b2):
    x_r = x.reshape(NW, NCH, CHUNK)
    pooled = _sc_pool(x_r, W_word, W_bi, W_tri)
    return _mlp(pooled, W1, b1, W2, b2)


# speedup vs baseline: 1.0594x; 1.0594x over previous
import functools

import jax
import jax.numpy as jnp
from jax import lax
from jax.experimental import pallas as pl
from jax.experimental.pallas import tpu as pltpu
from jax.experimental.pallas import tpu_sc as plsc

B = 4096
L = 50
D = 32
H = 128
C = 32

NC = 2
NS = 16
NW = NC * NS
BPW = B // NW
G = 2
CHUNK = G * L
NCH = BPW // G
INV_L = 1.0 / L


def _sc_pool(x_r, W_word, W_bi, W_tri):
    mesh = plsc.VectorSubcoreMesh(core_axis_name="c", subcore_axis_name="s")

    @functools.partial(
        pl.kernel,
        out_type=jax.ShapeDtypeStruct((B, 3 * D), jnp.float32),
        mesh=mesh,
        compiler_params=pltpu.CompilerParams(use_tc_tiling_on_sc=False),
        scratch_types=[
            pltpu.VMEM((NCH, CHUNK), jnp.int32),
            pltpu.VMEM((2, CHUNK, D), jnp.float32),
            pltpu.VMEM((2, CHUNK, D), jnp.float32),
            pltpu.VMEM((2, CHUNK, D), jnp.float32),
            pltpu.VMEM((BPW, 3 * D), jnp.float32),
            pltpu.SemaphoreType.DMA,
            pltpu.SemaphoreType.DMA,
            pltpu.SemaphoreType.DMA,
            pltpu.SemaphoreType.DMA,
            pltpu.SemaphoreType.DMA,
            pltpu.SemaphoreType.DMA,
        ],
    )
    def k(x_hbm, ww_hbm, wb_hbm, wt_hbm, out_hbm,
          idx_v, bufw, bufb, buft, pooled,
          sw0, sw1, sb0, sb1, st0, st1):
        wid = lax.axis_index("s") * NC + lax.axis_index("c")
        pltpu.sync_copy(x_hbm.at[wid], idx_v)

        tables = (ww_hbm, wb_hbm, wt_hbm)
        bufs = (bufw, bufb, buft)
        sems = ((sw0, sw1), (sb0, sb1), (st0, st1))

        def start(c, par):
            idx = idx_v.at[c]
            for t in range(3):
                pltpu.async_copy(tables[t].at[idx], bufs[t].at[par],
                                 sems[t][par])

        def wait(c, par):
            idx = idx_v.at[c]
            for t in range(3):
                pltpu.make_async_copy(tables[t].at[idx], bufs[t].at[par],
                                      sems[t][par]).wait()

        start(0, 0)

        def outer(i, carry):
            for par in range(2):
                c = 2 * i + par
                nc = jnp.minimum(c + 1, NCH - 1)
                start(nc, 1 - par)
                wait(c, par)
                for t in range(3):
                    buf = bufs[t].at[par]
                    for g in range(G):
                        a0 = buf[g * L, pl.ds(0, 16)]
                        a1 = buf[g * L, pl.ds(16, 16)]
                        for l in range(1, L):
                            a0 = a0 + buf[g * L + l, pl.ds(0, 16)]
                            a1 = a1 + buf[g * L + l, pl.ds(16, 16)]
                        row = c * G + g
                        pooled[row, pl.ds(t * D, 16)] = a0 * INV_L
                        pooled[row, pl.ds(t * D + 16, 16)] = a1 * INV_L
            return carry

        lax.fori_loop(0, NCH // 2, outer, 0)
        wait(NCH - 1, 0)

        pltpu.sync_copy(pooled, out_hbm.at[pl.ds(wid * BPW, BPW)])

    return k(x_r, W_word, W_bi, W_tri)


def _mlp(pooled, W1, b1, W2, b2):
    def body(p_ref, w1_ref, b1_ref, w2_ref, b2_ref, o_ref):
        h = jnp.dot(p_ref[...], w1_ref[...],
                    preferred_element_type=jnp.float32) + b1_ref[...]
        o = jnp.dot(h, w2_ref[...],
                    preferred_element_type=jnp.float32) + b2_ref[...]
        o_ref[...] = jnp.maximum(o, 0.0)

    return pl.pallas_call(
        body,
        out_shape=jax.ShapeDtypeStruct((B, C), jnp.float32),
    )(pooled, W1, b1.reshape(1, H), W2, b2.reshape(1, C))


def kernel(x, W_word, W_bi, W_tri, W1, b1, W2, b2):
    x_r = x.reshape(NW, NCH, CHUNK)
    pooled = _sc_pool(x_r, W_word, W_bi, W_tri)
    return _mlp(pooled, W1, b1, W2, b2)
